# baseline (speedup 1.0000x reference)
import functools

import jax
import jax.numpy as jnp
from jax import lax
from jax.experimental import pallas as pl
from jax.experimental.pallas import tpu as pltpu
from jax.experimental.pallas import tpu_sc as plsc


def _sc_window_copy(table, Q, K, offset):
    D = table.shape[1]
    info = plsc.get_sparse_core_info()
    NC = info.num_cores
    mesh = plsc.VectorSubcoreMesh(core_axis_name="c", subcore_axis_name="s")

    table_flat = table.reshape(-1)

    @functools.partial(
        pl.kernel,
        out_type=jax.ShapeDtypeStruct((Q, K * D), table.dtype),
        mesh=mesh,
    )
    def copy_kernel(table_hbm, out_hbm):
        wid = lax.axis_index("s") * NC + lax.axis_index("c")
        start = (offset - wid) * D
        pltpu.sync_copy(table_hbm.at[pl.ds(start, K * D)], out_hbm.at[wid])

    return copy_kernel(table_flat).reshape(Q, K, D)


def kernel(q, k, embed_weight):
    Q = q.shape[0]
    K = k.shape[0]
    max_len = embed_weight.shape[0]
    offset = max_len // 2 + max_len % 2
    return _sc_window_copy(embed_weight, Q, K, offset)
